# baseline (speedup 1.0000x reference)
import functools

import jax
import jax.numpy as jnp
from jax import lax
from jax.experimental import pallas as pl
from jax.experimental.pallas import tpu as pltpu
from jax.experimental.pallas import tpu_sc as plsc

N = 16777
D = 128
K9 = 9
KC = 4
NC, NS, L = 2, 16, 16
NW = NC * NS
RPW = 544
NPAD = NW * RPW
CAP = 8 * RPW
GB = 128
CH1 = RPW // 4

_sc_params = pltpu.CompilerParams(needs_layout_passes=False)


@functools.lru_cache(maxsize=1)
def _mesh():
    return plsc.VectorSubcoreMesh(core_axis_name="c", subcore_axis_name="s")


def _sc_gather1_body(xp_hbm, gidx_hbm, out_hbm, x_v, idx_v, g_v, semx, semo):
    wid = lax.axis_index("s") * NC + lax.axis_index("c")
    pltpu.async_copy(xp_hbm, x_v, semx).wait()
    oh = {}
    for t in range(4):
        s = t % 2
        if t == 0:
            pltpu.sync_copy(gidx_hbm.at[wid, 0], idx_v.at[0])

        def body(m, _, s=s):
            iv = idx_v[s, pl.ds(m * L, L)]
            g_v[s, pl.ds(m * L, L)] = plsc.load_gather(x_v, [iv])
            return 0
        lax.fori_loop(0, (CH1 * 128) // L, body, 0)
        if t + 1 < 4:
            pltpu.sync_copy(gidx_hbm.at[wid, t + 1], idx_v.at[1 - s])
        if t >= 2:
            oh[s].wait()
        row0 = (wid * RPW + t * CH1) * 128
        oh[s] = pltpu.async_copy(g_v.at[s], out_hbm.at[pl.ds(row0, CH1 * 128)], semo.at[s])
    oh[0].wait()
    oh[1].wait()


def _sc_gather1(xp, gidx):
    fn = pl.kernel(
        _sc_gather1_body,
        mesh=_mesh(),
        out_type=jax.ShapeDtypeStruct((NPAD * 128,), jnp.float32),
        scratch_types=[
            pltpu.VMEM((NPAD,), jnp.float32),
            pltpu.VMEM((2, CH1 * 128), jnp.int32),
            pltpu.VMEM((2, CH1 * 128), jnp.float32),
            pltpu.SemaphoreType.DMA,
            pltpu.SemaphoreType.DMA((2,)),
        ],
        compiler_params=_sc_params,
    )
    return fn(xp, gidx)


def _sc_corr_body(ztab_hbm, src_hbm, dst_hbm, cnt_hbm, out_hbm,
                  acc_v, buf_v, idx_v, dst_v, cnt_v, semg):
    wid = lax.axis_index("s") * NC + lax.axis_index("c")
    pltpu.sync_copy(cnt_hbm, cnt_v)
    cnt = cnt_v[pl.ds(wid, L)][0]

    zero = jnp.zeros((L,), jnp.float32)

    def zbody(r, _):
        for j in range(D // L):
            acc_v[r, pl.ds(j * L, L)] = zero
        return 0
    lax.fori_loop(0, RPW, zbody, 0)

    nb = lax.div(cnt + (GB - 1), GB)

    def batch(b, _):
        pltpu.sync_copy(src_hbm.at[wid, pl.ds(b * GB, GB)], idx_v)
        pltpu.sync_copy(dst_hbm.at[wid, pl.ds(b * GB, GB)], dst_v.at[pl.ds(0, GB)])
        pltpu.async_copy(ztab_hbm.at[idx_v], buf_v, semg).wait()
        lim = jnp.minimum(GB, cnt - b * GB)

        def mbody(m, _):
            d = dst_v[pl.ds(m, L)][0]
            for j in range(D // L):
                acc_v[d, pl.ds(j * L, L)] = acc_v[d, pl.ds(j * L, L)] + buf_v[m, pl.ds(j * L, L)]
            return 0
        lax.fori_loop(0, lim, mbody, 0)
        return 0
    lax.fori_loop(0, nb, batch, 0)

    pltpu.sync_copy(acc_v, out_hbm.at[pl.ds(wid * RPW, RPW)])


def _sc_corr(ztab, src, dst, cnt):
    fn = pl.kernel(
        _sc_corr_body,
        mesh=_mesh(),
        out_type=jax.ShapeDtypeStruct((NPAD, D), jnp.float32),
        scratch_types=[
            pltpu.VMEM((RPW, D), jnp.float32),
            pltpu.VMEM((GB, D), jnp.float32),
            pltpu.VMEM((GB,), jnp.int32),
            pltpu.VMEM((GB + L,), jnp.int32),
            pltpu.VMEM((NW + L,), jnp.int32),
            pltpu.SemaphoreType.DMA,
        ],
        compiler_params=_sc_params,
    )
    return fn(ztab, src, dst, cnt)


def _mm_body(a_ref, w_ref, o_ref, *, relu_in):
    a = a_ref[...]
    if relu_in:
        a = jnp.maximum(a, 0.0)
    o_ref[...] = jnp.dot(a, w_ref[...], preferred_element_type=jnp.float32)


def _tc_matmul(a, w, relu_in, bm=1024):
    m, kin = a.shape
    kout = w.shape[1]
    return pl.pallas_call(
        functools.partial(_mm_body, relu_in=relu_in),
        grid=(m // bm,),
        in_specs=[
            pl.BlockSpec((bm, kin), lambda i: (i, 0)),
            pl.BlockSpec((kin, kout), lambda i: (0, 0)),
        ],
        out_specs=pl.BlockSpec((bm, kout), lambda i: (i, 0)),
        out_shape=jax.ShapeDtypeStruct((m, kout), jnp.float32),
    )(a, w)


def _mm_pair_body(z_ref, c_ref, w_ref, o_ref):
    a = jnp.maximum(z_ref[...] + c_ref[...], 0.0)
    o_ref[...] = jnp.dot(a, w_ref[...], preferred_element_type=jnp.float32)


def _tc_matmul_pair(z, c, w, bm=1024):
    kout = w.shape[1]
    return pl.pallas_call(
        _mm_pair_body,
        grid=(NPAD // bm,),
        in_specs=[
            pl.BlockSpec((bm, D), lambda i: (i, KC)),
            pl.BlockSpec((bm, D), lambda i: (i, 0)),
            pl.BlockSpec((D, kout), lambda i: (0, 0)),
        ],
        out_specs=pl.BlockSpec((bm, kout), lambda i: (i, 0)),
        out_shape=jax.ShapeDtypeStruct((NPAD, kout), jnp.float32),
    )(z, c, w)


def _head_body(s1, z2, c2, z3, c3, z4, c4, wf1, bf1, wf2, bf2, o_ref, acc):
    i = pl.program_id(0)

    @pl.when(i == 0)
    def _init():
        acc[...] = jnp.zeros_like(acc)

    ys = (s1[...], z2[...] + c2[...], z3[...] + c3[...], z4[...] + c4[...])
    for idx, y in enumerate(ys):
        acc[0:1, idx * D:(idx + 1) * D] += jnp.sum(jnp.maximum(y, 0.0), axis=0, keepdims=True)

    @pl.when(i == pl.num_programs(0) - 1)
    def _final():
        p = acc[...] * (1.0 / N)
        h = jnp.maximum(
            jnp.dot(p, wf1[...], preferred_element_type=jnp.float32) + bf1[...], 0.0
        )
        o_ref[...] = jnp.dot(h, wf2[...], preferred_element_type=jnp.float32) + bf2[...]


def _tc_head(s1, zc2, zc3, zc4, wf1, bf1, wf2, bf2, bm=1024):
    sspec = pl.BlockSpec((bm, D), lambda i: (i, 0))
    zspec = pl.BlockSpec((bm, D), lambda i: (i, KC))
    full = lambda shape: pl.BlockSpec(shape, lambda i: (0, 0))
    return pl.pallas_call(
        _head_body,
        grid=(NPAD // bm,),
        in_specs=[sspec, zspec, sspec, zspec, sspec, zspec, sspec,
                  full((512, 512)), full((1, 512)), full((512, D)), full((1, D))],
        out_specs=full((1, D)),
        out_shape=jax.ShapeDtypeStruct((1, D), jnp.float32),
        scratch_shapes=[pltpu.VMEM((1, 512), jnp.float32)],
    )(s1, zc2[0], zc2[1], zc3[0], zc3[1], zc4[0], zc4[1], wf1, bf1, wf2, bf2)


def _corr_lists(neigh):
    ni = neigh.astype(jnp.int32)
    mask = (ni != N).at[:, KC].set(False)
    mflat = mask.reshape(-1)
    iflat = jnp.broadcast_to(jnp.arange(N, dtype=jnp.int32)[:, None], (N, K9)).reshape(-1)
    sflat = (ni * K9 + jnp.arange(K9, dtype=jnp.int32)[None, :]).reshape(-1)
    e = jnp.concatenate([jnp.zeros((1,), jnp.int32),
                         jnp.cumsum(mflat.astype(jnp.int32))])
    w = iflat // RPW
    base = e[w * (RPW * K9)]
    rank = e[: N * K9] - base
    pos = jnp.where(mflat, w * CAP + rank, NW * CAP)
    src = jnp.zeros((NW * CAP + 1,), jnp.int32).at[pos].set(sflat).at[NW * CAP].set(0)
    dst = jnp.zeros((NW * CAP + 1,), jnp.int32).at[pos].set(iflat - w * RPW).at[NW * CAP].set(0)
    bounds = jnp.arange(NW + 1, dtype=jnp.int32) * (RPW * K9)
    bounds = jnp.minimum(bounds, N * K9)
    eb = e[bounds]
    cnt = jnp.concatenate([eb[1:] - eb[:-1], jnp.zeros((L,), jnp.int32)])
    return src[:NW * CAP].reshape(NW, CAP), dst[:NW * CAP].reshape(NW, CAP), cnt


def kernel(x, neigh5, neigh3d1, neigh3d2, neigh3d3, W5, W2a, W2b, W2c,
           W3a, W3b, W3c, W4a, W4b, W4c, Wf1, bf1, Wf2, bf2):
    xp = jnp.zeros((NPAD,), jnp.float32).at[:N].set(x[:, 0])
    g1 = jnp.full((NPAD, 128), N, jnp.int32).at[:N, :25].set(neigh5.astype(jnp.int32))
    g1 = g1.reshape(NW, 4, CH1 * 128)
    W5p = jnp.zeros((128, 128), jnp.float32).at[:25].set(W5[:, 0, :])

    G = _sc_gather1(xp, g1).reshape(NPAD, 128)
    S1 = _tc_matmul(G, W5p, relu_in=False)

    corr = [_corr_lists(n) for n in (neigh3d1, neigh3d2, neigh3d3)]
    Ws = [W2a, W2b, W2c, W3a, W3b, W3c, W4a, W4b, W4c]
    pooled = []
    zc = None
    for li in range(9):
        Wcat = Ws[li].transpose(1, 0, 2).reshape(D, K9 * D)
        if li == 0:
            Z = _tc_matmul(S1, Wcat, relu_in=True)
        else:
            Z = _tc_matmul_pair(zc[0], zc[1], Wcat)
        src, dst, cnt = corr[li % 3]
        C = _sc_corr(Z.reshape(NPAD * K9, D), src, dst, cnt)
        zc = (Z, C)
        if li in (2, 5, 8):
            pooled.append(zc)

    return _tc_head(S1, pooled[0], pooled[1], pooled[2],
                    Wf1, bf1.reshape(1, 512), Wf2, bf2.reshape(1, D))


# speedup vs baseline: 15.7235x; 15.7235x over previous
import functools

import jax
import jax.numpy as jnp
from jax import lax
from jax.experimental import pallas as pl
from jax.experimental.pallas import tpu as pltpu
from jax.experimental.pallas import tpu_sc as plsc

N = 16777
D = 128
K9 = 9
KC = 4
K25 = 25
KC5 = 12
NC, NS, L = 2, 16, 16
NW = NC * NS
RPW = 544
NPAD = NW * RPW
CAP = 8 * RPW
CAP5 = 24 * RPW
PBIG = 1 << 30
LROW = CAP + 5 * L
LROW5 = CAP5 + 5 * L
CHW = 64
WV = 16
G9 = (RPW * K9) // L
G25 = (RPW * K25) // L

_sc_params = pltpu.CompilerParams(needs_layout_passes=False)


@functools.lru_cache(maxsize=1)
def _mesh():
    return plsc.VectorSubcoreMesh(core_axis_name="c", subcore_axis_name="s")


def _nsrc9(neigh):
    ni = jnp.full((NPAD, K9), N, jnp.int32).at[:N].set(neigh.astype(jnp.int32))
    karr = jnp.arange(K9, dtype=jnp.int32)[None, :]
    k8 = jnp.where(karr < KC, karr, karr - 1)
    dloc = (jnp.arange(NPAD, dtype=jnp.int32) % RPW)[:, None]
    v = jnp.where((ni != N) & (karr != KC),
                  (((ni << 3) | k8) << 10) | dloc, PBIG)
    return v.reshape(-1)


def _nsrc25(neigh5):
    ni = jnp.full((NPAD, K25), N, jnp.int32).at[:N].set(neigh5.astype(jnp.int32))
    karr = jnp.arange(K25, dtype=jnp.int32)[None, :]
    dloc = (jnp.arange(NPAD, dtype=jnp.int32) % RPW)[:, None]
    v = jnp.where((ni != N) & (karr != KC5),
                  (((ni << 5) | karr) << 10) | dloc, PBIG)
    return v.reshape(-1)


def _sc_compact_body(nsrcp_hbm, lists_hbm, nsrc_v, list_v):
    wid = lax.axis_index("s") * NC + lax.axis_index("c")
    zi = jnp.zeros((L,), jnp.int32)

    def one(src_off, ngroups, cap, dst_off):
        pltpu.sync_copy(nsrcp_hbm.at[pl.ds(src_off, ngroups * L)],
                        nsrc_v.at[pl.ds(0, ngroups * L)])

        def cbody(g, off):
            sv = nsrc_v[pl.ds(g * L, L)]
            mask = sv < PBIG
            plsc.store_compressed(list_v.at[pl.ds(off, L)], sv, mask=mask)
            return off + plsc.all_reduce_population_count(mask)[0]
        cnt = lax.fori_loop(0, ngroups, cbody, jnp.int32(0))
        for t in range(4):
            list_v[pl.ds(cnt + t * L, L)] = zi
        list_v[pl.ds(cap + 4 * L, L)] = jnp.full((L,), cnt, jnp.int32)
        pltpu.sync_copy(list_v.at[pl.ds(0, cap + 5 * L)],
                        lists_hbm.at[pl.ds(dst_off, cap + 5 * L)])

    for q in range(3):
        one((q * NW + wid) * (RPW * K9), G9, CAP, (q * NW + wid) * LROW)
    one(3 * NW * (RPW * K9) + wid * (RPW * K25), G25, CAP5,
        3 * NW * LROW + wid * LROW5)


def _sc_compact(nsrcp):
    fn = pl.kernel(
        _sc_compact_body,
        mesh=_mesh(),
        out_type=jax.ShapeDtypeStruct((3 * NW * LROW + NW * LROW5,), jnp.int32),
        scratch_types=[
            pltpu.VMEM((RPW * K25,), jnp.int32),
            pltpu.VMEM((LROW5,), jnp.int32),
        ],
        compiler_params=_sc_params,
    )
    return fn(nsrcp)


def _make_sc_corr_body(q):
    def body(zc_hbm, c_hbm, w8_hbm, lists_hbm, out_hbm,
             acc_v, wtap_v, rowz_v, rowc_v, yrow_v, chunk_v, cntv_v,
             ksm, semr, semt):
        wid = lax.axis_index("s") * NC + lax.axis_index("c")
        base = (q * NW + wid) * LROW
        pltpu.sync_copy(lists_hbm.at[pl.ds(base + CAP + 4 * L, L)], cntv_v)
        zero = jnp.zeros((L,), jnp.float32)

        def zbody(r, _):
            for j in range(D // L):
                acc_v[r, pl.ds(j * L, L)] = zero
            return 0
        lax.fori_loop(0, RPW, zbody, 0)

        ksm[0] = jnp.int32(-1)
        cnt = cntv_v[pl.ds(0, L)][0]
        nch = lax.div(cnt + (CHW - 1), CHW)

        def chunk(ch, _):
            pltpu.sync_copy(lists_hbm.at[pl.ds(base + ch * CHW, CHW)],
                            chunk_v.at[pl.ds(0, CHW)])
            rem = cnt - ch * CHW
            for wv in range(CHW // WV):
                @pl.when(wv * WV < rem)
                def _wave(wv=wv):
                    hs = []
                    for u in range(WV):
                        v = chunk_v[pl.ds(wv * WV + u, L)][0]
                        j = lax.shift_right_logical(v, 13)
                        hs.append(pltpu.async_copy(
                            zc_hbm.at[pl.ds(j, 1)], rowz_v.at[pl.ds(u, 1)], semr))
                        hs.append(pltpu.async_copy(
                            c_hbm.at[pl.ds(j, 1)], rowc_v.at[pl.ds(u, 1)], semr))
                    for h in hs:
                        h.wait()
                    lim = jnp.minimum(WV, rem - wv * WV)

                    def mbody(u, _):
                        v = chunk_v[pl.ds(wv * WV + u, L)][0]
                        k8 = lax.bitwise_and(lax.shift_right_logical(v, 10), 7)
                        d = lax.bitwise_and(v, 1023)

                        @pl.when(k8 != ksm[0])
                        def _load_tap():
                            pltpu.async_copy(
                                w8_hbm.at[pl.ds(k8 * D, D)], wtap_v, semt).wait()
                            ksm[0] = k8

                        for jj in range(D // L):
                            y = jnp.maximum(
                                rowz_v[u, pl.ds(jj * L, L)]
                                + rowc_v[u, pl.ds(jj * L, L)], 0.0)
                            yrow_v[pl.ds(jj * L, L)] = y

                        def cvec(cc, accs):
                            xs = yrow_v[pl.ds(cc, L)][0]
                            xv = jnp.full((L,), xs, jnp.float32)
                            return tuple(
                                accs[jj] + xv * wtap_v[cc, pl.ds(jj * L, L)]
                                for jj in range(D // L))
                        accs = lax.fori_loop(
                            0, D, cvec,
                            tuple(jnp.zeros((L,), jnp.float32)
                                  for _ in range(D // L)))
                        for jj in range(D // L):
                            acc_v[d, pl.ds(jj * L, L)] = (
                                acc_v[d, pl.ds(jj * L, L)] + accs[jj])
                        return 0
                    lax.fori_loop(0, lim, mbody, 0)
            return 0
        lax.fori_loop(0, nch, chunk, 0)

        pltpu.sync_copy(acc_v, out_hbm.at[pl.ds(wid * RPW, RPW)])
    return body


def _sc_corr(zc, cprev, w8, lists, q):
    fn = pl.kernel(
        _make_sc_corr_body(q),
        mesh=_mesh(),
        out_type=jax.ShapeDtypeStruct((NPAD, D), jnp.float32),
        scratch_types=[
            pltpu.VMEM((RPW, D), jnp.float32),
            pltpu.VMEM((D, D), jnp.float32),
            pltpu.VMEM((WV, D), jnp.float32),
            pltpu.VMEM((WV + 1, D), jnp.float32),
            pltpu.VMEM((D + L,), jnp.float32),
            pltpu.VMEM((CHW + L,), jnp.int32),
            pltpu.VMEM((L,), jnp.int32),
            pltpu.SMEM((8,), jnp.int32),
            pltpu.SemaphoreType.DMA,
            pltpu.SemaphoreType.DMA,
        ],
        compiler_params=_sc_params,
    )
    return fn(zc, cprev, w8, lists)


def _sc_corr1_body(xp_hbm, w5_hbm, lists_hbm, out_hbm,
                   acc_v, x_v, w5_v, chunk_v, cntv_v, semx):
    wid = lax.axis_index("s") * NC + lax.axis_index("c")
    base = 3 * NW * LROW + wid * LROW5
    pltpu.sync_copy(lists_hbm.at[pl.ds(base + CAP5 + 4 * L, L)], cntv_v)
    pltpu.async_copy(xp_hbm, x_v, semx).wait()
    pltpu.sync_copy(w5_hbm, w5_v)
    zero = jnp.zeros((L,), jnp.float32)

    def zbody(r, _):
        for j in range(D // L):
            acc_v[r, pl.ds(j * L, L)] = zero
        return 0
    lax.fori_loop(0, RPW, zbody, 0)

    cnt = cntv_v[pl.ds(0, L)][0]
    nch = lax.div(cnt + (CHW - 1), CHW)

    def chunk(ch, _):
        pltpu.sync_copy(lists_hbm.at[pl.ds(base + ch * CHW, CHW)],
                        chunk_v.at[pl.ds(0, CHW)])
        lim = jnp.minimum(CHW, cnt - ch * CHW)

        def mbody(m, _):
            v = chunk_v[pl.ds(m, L)][0]
            j = lax.shift_right_logical(v, 15)
            k = lax.bitwise_and(lax.shift_right_logical(v, 10), 31)
            d = lax.bitwise_and(v, 1023)
            xs = x_v[pl.ds(j, L)][0]
            xv = jnp.full((L,), xs, jnp.float32)
            for jj in range(D // L):
                acc_v[d, pl.ds(jj * L, L)] = (
                    acc_v[d, pl.ds(jj * L, L)] + xv * w5_v[k, pl.ds(jj * L, L)])
            return 0
        lax.fori_loop(0, lim, mbody, 0)
        return 0
    lax.fori_loop(0, nch, chunk, 0)

    pltpu.sync_copy(acc_v, out_hbm.at[pl.ds(wid * RPW, RPW)])


def _sc_corr1(xp, w5, lists):
    fn = pl.kernel(
        _sc_corr1_body,
        mesh=_mesh(),
        out_type=jax.ShapeDtypeStruct((NPAD, D), jnp.float32),
        scratch_types=[
            pltpu.VMEM((RPW, D), jnp.float32),
            pltpu.VMEM((NPAD,), jnp.float32),
            pltpu.VMEM((K25, D), jnp.float32),
            pltpu.VMEM((CHW + L,), jnp.int32),
            pltpu.VMEM((L,), jnp.int32),
            pltpu.SemaphoreType.DMA,
        ],
        compiler_params=_sc_params,
    )
    return fn(xp, w5, lists)


def _outer_body(x_ref, w5c_ref, o_ref):
    o_ref[...] = x_ref[...] * w5c_ref[...]


def _tc_outer(x2, w5c, bm=1024):
    return pl.pallas_call(
        _outer_body,
        grid=(NPAD // bm,),
        in_specs=[
            pl.BlockSpec((bm, 1), lambda i: (i, 0)),
            pl.BlockSpec((1, D), lambda i: (0, 0)),
        ],
        out_specs=pl.BlockSpec((bm, D), lambda i: (i, 0)),
        out_shape=jax.ShapeDtypeStruct((NPAD, D), jnp.float32),
    )(x2, w5c)


def _conv_body(z_ref, c_ref, wc_ref, o_ref):
    a = jnp.maximum(z_ref[...] + c_ref[...], 0.0)
    o_ref[...] = jnp.dot(a, wc_ref[...], preferred_element_type=jnp.float32)


def _tc_conv(zc, c, wc, bm=1024):
    return pl.pallas_call(
        _conv_body,
        grid=(NPAD // bm,),
        in_specs=[
            pl.BlockSpec((bm, D), lambda i: (i, 0)),
            pl.BlockSpec((bm, D), lambda i: (i, 0)),
            pl.BlockSpec((D, D), lambda i: (0, 0)),
        ],
        out_specs=pl.BlockSpec((bm, D), lambda i: (i, 0)),
        out_shape=jax.ShapeDtypeStruct((NPAD, D), jnp.float32),
    )(zc, c, wc)


def _head_body(z1, c1, z2, c2, z3, c3, z4, c4,
               wf1, bf1, wf2, bf2, o_ref, acc):
    i = pl.program_id(0)

    @pl.when(i == 0)
    def _init():
        acc[...] = jnp.zeros_like(acc)

    ys = (z1[...] + c1[...],
          z2[...] + c2[...], z3[...] + c3[...], z4[...] + c4[...])
    for idx, y in enumerate(ys):
        acc[0:1, idx * D:(idx + 1) * D] += jnp.sum(jnp.maximum(y, 0.0), axis=0, keepdims=True)

    @pl.when(i == pl.num_programs(0) - 1)
    def _final():
        p = acc[...] * (1.0 / N)
        h = jnp.maximum(
            jnp.dot(p, wf1[...], preferred_element_type=jnp.float32) + bf1[...], 0.0
        )
        o_ref[...] = jnp.dot(h, wf2[...], preferred_element_type=jnp.float32) + bf2[...]


def _tc_head(zc1, zc2, zc3, zc4, wf1, bf1, wf2, bf2, bm=1024):
    sspec = pl.BlockSpec((bm, D), lambda i: (i, 0))
    full = lambda shape: pl.BlockSpec(shape, lambda i: (0, 0))
    return pl.pallas_call(
        _head_body,
        grid=(NPAD // bm,),
        in_specs=[sspec, sspec, sspec, sspec, sspec, sspec, sspec, sspec,
                  full((512, 512)), full((1, 512)), full((512, D)), full((1, D))],
        out_specs=full((1, D)),
        out_shape=jax.ShapeDtypeStruct((1, D), jnp.float32),
        scratch_shapes=[pltpu.VMEM((1, 512), jnp.float32)],
    )(zc1[0], zc1[1], zc2[0], zc2[1], zc3[0], zc3[1], zc4[0], zc4[1],
      wf1, bf1, wf2, bf2)


def kernel(x, neigh5, neigh3d1, neigh3d2, neigh3d3, W5, W2a, W2b, W2c,
           W3a, W3b, W3c, W4a, W4b, W4c, Wf1, bf1, Wf2, bf2):
    xp = jnp.zeros((NPAD,), jnp.float32).at[:N].set(x[:, 0])
    x2 = xp.reshape(NPAD, 1)
    w5mat = W5[:, 0, :]
    w5c = w5mat[KC5:KC5 + 1]

    nsrcp = jnp.concatenate(
        [_nsrc9(n) for n in (neigh3d1, neigh3d2, neigh3d3)] + [_nsrc25(neigh5)])
    lists = _sc_compact(nsrcp)

    C1 = _sc_corr1(xp, w5mat, lists)
    Zc1 = _tc_outer(x2, w5c)

    Ws = [W2a, W2b, W2c, W3a, W3b, W3c, W4a, W4b, W4c]
    pooled = [(Zc1, C1)]
    zc = (Zc1, C1)
    for li in range(9):
        wl = Ws[li]
        wc = wl[KC]
        w8 = jnp.concatenate([wl[k] for k in range(K9) if k != KC], axis=0)
        Z = _tc_conv(zc[0], zc[1], wc)
        C = _sc_corr(zc[0], zc[1], w8, lists, li % 3)
        zc = (Z, C)
        if li in (2, 5, 8):
            pooled.append(zc)

    return _tc_head(pooled[0], pooled[1], pooled[2], pooled[3],
                    Wf1, bf1.reshape(1, 512), Wf2, bf2.reshape(1, D))


# speedup vs baseline: 17.6027x; 1.1195x over previous
import functools

import jax
import jax.numpy as jnp
from jax import lax
from jax.experimental import pallas as pl
from jax.experimental.pallas import tpu as pltpu
from jax.experimental.pallas import tpu_sc as plsc

N = 16777
D = 128
K9 = 9
KC = 4
K25 = 25
KC5 = 12
NC, NS, L = 2, 16, 16
NW = NC * NS
RPW = 544
NPAD = NW * RPW
CAP = 8 * RPW
CAP5 = 24 * RPW
PBIG = 1 << 30
LROW = CAP + 5 * L
LROW5 = CAP5 + 5 * L
CHW = 64
WV = 16
G9 = (RPW * K9) // L
G25 = (RPW * K25) // L

_sc_params = pltpu.CompilerParams(needs_layout_passes=False)


@functools.lru_cache(maxsize=1)
def _mesh():
    return plsc.VectorSubcoreMesh(core_axis_name="c", subcore_axis_name="s")


def _nsrc9(neigh):
    ni = jnp.full((NPAD, K9), N, jnp.int32).at[:N].set(neigh.astype(jnp.int32))
    karr = jnp.arange(K9, dtype=jnp.int32)[None, :]
    k8 = jnp.where(karr < KC, karr, karr - 1)
    dloc = (jnp.arange(NPAD, dtype=jnp.int32) % RPW)[:, None]
    v = jnp.where((ni != N) & (karr != KC),
                  (((ni << 3) | k8) << 10) | dloc, PBIG)
    return v.reshape(NW, RPW, K9).transpose(0, 2, 1).reshape(-1)


def _nsrc25(neigh5):
    ni = jnp.full((NPAD, K25), N, jnp.int32).at[:N].set(neigh5.astype(jnp.int32))
    karr = jnp.arange(K25, dtype=jnp.int32)[None, :]
    dloc = (jnp.arange(NPAD, dtype=jnp.int32) % RPW)[:, None]
    v = jnp.where((ni != N) & (karr != KC5),
                  (((ni << 5) | karr) << 10) | dloc, PBIG)
    return v.reshape(-1)


def _sc_compact_body(nsrcp_hbm, lists_hbm, nsrc_v, list_v):
    wid = lax.axis_index("s") * NC + lax.axis_index("c")
    zi = jnp.zeros((L,), jnp.int32)

    def one(src_off, ngroups, cap, dst_off):
        pltpu.sync_copy(nsrcp_hbm.at[pl.ds(src_off, ngroups * L)],
                        nsrc_v.at[pl.ds(0, ngroups * L)])

        def cbody(g, off):
            sv = nsrc_v[pl.ds(g * L, L)]
            mask = sv < PBIG
            plsc.store_compressed(list_v.at[pl.ds(off, L)], sv, mask=mask)
            return off + plsc.all_reduce_population_count(mask)[0]
        cnt = lax.fori_loop(0, ngroups, cbody, jnp.int32(0))
        for t in range(4):
            list_v[pl.ds(cnt + t * L, L)] = zi
        list_v[pl.ds(cap + 4 * L, L)] = jnp.full((L,), cnt, jnp.int32)
        pltpu.sync_copy(list_v.at[pl.ds(0, cap + 5 * L)],
                        lists_hbm.at[pl.ds(dst_off, cap + 5 * L)])

    for q in range(3):
        one((q * NW + wid) * (RPW * K9), G9, CAP, (q * NW + wid) * LROW)
    one(3 * NW * (RPW * K9) + wid * (RPW * K25), G25, CAP5,
        3 * NW * LROW + wid * LROW5)


def _sc_compact(nsrcp):
    fn = pl.kernel(
        _sc_compact_body,
        mesh=_mesh(),
        out_type=jax.ShapeDtypeStruct((3 * NW * LROW + NW * LROW5,), jnp.int32),
        scratch_types=[
            pltpu.VMEM((RPW * K25,), jnp.int32),
            pltpu.VMEM((LROW5,), jnp.int32),
        ],
        compiler_params=_sc_params,
    )
    return fn(nsrcp)


def _make_sc_corr_body(q):
    def body(zc_hbm, c_hbm, w8_hbm, lists_hbm, out_hbm,
             acc_v, wtap_v, rowz_v, rowc_v, yrow_v, chunk_v, cntv_v,
             ksm, semr, semt):
        wid = lax.axis_index("s") * NC + lax.axis_index("c")
        base = (q * NW + wid) * LROW
        pltpu.sync_copy(lists_hbm.at[pl.ds(base + CAP + 4 * L, L)], cntv_v)
        zero = jnp.zeros((L,), jnp.float32)

        def zbody(r, _):
            for j in range(D // L):
                acc_v[r, pl.ds(j * L, L)] = zero
            return 0
        lax.fori_loop(0, RPW, zbody, 0)

        ksm[0] = jnp.int32(-1)
        cnt = cntv_v[pl.ds(0, L)][0]
        nch = lax.div(cnt + (CHW - 1), CHW)

        def chunk(ch, _):
            pltpu.sync_copy(lists_hbm.at[pl.ds(base + ch * CHW, CHW)],
                            chunk_v.at[pl.ds(0, CHW)])
            rem = cnt - ch * CHW
            for wv in range(CHW // WV):
                @pl.when(wv * WV < rem)
                def _wave(wv=wv):
                    hs = []
                    for u in range(WV):
                        v = chunk_v[pl.ds(wv * WV + u, L)][0]
                        j = lax.shift_right_logical(v, 13)
                        hs.append(pltpu.async_copy(
                            zc_hbm.at[pl.ds(j, 1)], rowz_v.at[pl.ds(u, 1)], semr))
                        hs.append(pltpu.async_copy(
                            c_hbm.at[pl.ds(j, 1)], rowc_v.at[pl.ds(u, 1)], semr))
                    for h in hs:
                        h.wait()
                    lim = jnp.minimum(WV, rem - wv * WV)

                    def mbody(u, _):
                        v = chunk_v[pl.ds(wv * WV + u, L)][0]
                        k8 = lax.bitwise_and(lax.shift_right_logical(v, 10), 7)
                        d = lax.bitwise_and(v, 1023)

                        @pl.when(k8 != ksm[0])
                        def _load_tap():
                            pltpu.async_copy(
                                w8_hbm.at[pl.ds(k8 * D, D)], wtap_v, semt).wait()
                            ksm[0] = k8

                        for jj in range(D // L):
                            y = jnp.maximum(
                                rowz_v[u, pl.ds(jj * L, L)]
                                + rowc_v[u, pl.ds(jj * L, L)], 0.0)
                            yrow_v[pl.ds(jj * L, L)] = y

                        def cvec(c4, accs):
                            for t in range(4):
                                cc = c4 * 4 + t
                                xs = yrow_v[pl.ds(cc, L)][0]
                                xv = jnp.full((L,), xs, jnp.float32)
                                accs = tuple(
                                    accs[jj] + xv * wtap_v[cc, pl.ds(jj * L, L)]
                                    for jj in range(D // L))
                            return accs
                        accs = lax.fori_loop(
                            0, D // 4, cvec,
                            tuple(jnp.zeros((L,), jnp.float32)
                                  for _ in range(D // L)))
                        for jj in range(D // L):
                            acc_v[d, pl.ds(jj * L, L)] = (
                                acc_v[d, pl.ds(jj * L, L)] + accs[jj])
                        return 0
                    lax.fori_loop(0, lim, mbody, 0)
            return 0
        lax.fori_loop(0, nch, chunk, 0)

        pltpu.sync_copy(acc_v, out_hbm.at[pl.ds(wid * RPW, RPW)])
    return body


def _sc_corr(zc, cprev, w8, lists, q):
    fn = pl.kernel(
        _make_sc_corr_body(q),
        mesh=_mesh(),
        out_type=jax.ShapeDtypeStruct((NPAD, D), jnp.float32),
        scratch_types=[
            pltpu.VMEM((RPW, D), jnp.float32),
            pltpu.VMEM((D, D), jnp.float32),
            pltpu.VMEM((WV, D), jnp.float32),
            pltpu.VMEM((WV + 1, D), jnp.float32),
            pltpu.VMEM((D + L,), jnp.float32),
            pltpu.VMEM((CHW + L,), jnp.int32),
            pltpu.VMEM((L,), jnp.int32),
            pltpu.SMEM((8,), jnp.int32),
            pltpu.SemaphoreType.DMA,
            pltpu.SemaphoreType.DMA,
        ],
        compiler_params=_sc_params,
    )
    return fn(zc, cprev, w8, lists)


def _sc_corr1_body(xp_hbm, w5_hbm, lists_hbm, out_hbm,
                   acc_v, x_v, w5_v, chunk_v, cntv_v, semx):
    wid = lax.axis_index("s") * NC + lax.axis_index("c")
    base = 3 * NW * LROW + wid * LROW5
    pltpu.sync_copy(lists_hbm.at[pl.ds(base + CAP5 + 4 * L, L)], cntv_v)
    pltpu.async_copy(xp_hbm, x_v, semx).wait()
    pltpu.sync_copy(w5_hbm, w5_v)
    zero = jnp.zeros((L,), jnp.float32)

    def zbody(r, _):
        for j in range(D // L):
            acc_v[r, pl.ds(j * L, L)] = zero
        return 0
    lax.fori_loop(0, RPW, zbody, 0)

    cnt = cntv_v[pl.ds(0, L)][0]
    nch = lax.div(cnt + (CHW - 1), CHW)

    def chunk(ch, _):
        pltpu.sync_copy(lists_hbm.at[pl.ds(base + ch * CHW, CHW)],
                        chunk_v.at[pl.ds(0, CHW)])
        lim = jnp.minimum(CHW, cnt - ch * CHW)

        def mbody(m, _):
            v = chunk_v[pl.ds(m, L)][0]
            j = lax.shift_right_logical(v, 15)
            k = lax.bitwise_and(lax.shift_right_logical(v, 10), 31)
            d = lax.bitwise_and(v, 1023)
            xs = x_v[pl.ds(j, L)][0]
            xv = jnp.full((L,), xs, jnp.float32)
            for jj in range(D // L):
                acc_v[d, pl.ds(jj * L, L)] = (
                    acc_v[d, pl.ds(jj * L, L)] + xv * w5_v[k, pl.ds(jj * L, L)])
            return 0
        lax.fori_loop(0, lim, mbody, 0)
        return 0
    lax.fori_loop(0, nch, chunk, 0)

    pltpu.sync_copy(acc_v, out_hbm.at[pl.ds(wid * RPW, RPW)])


def _sc_corr1(xp, w5, lists):
    fn = pl.kernel(
        _sc_corr1_body,
        mesh=_mesh(),
        out_type=jax.ShapeDtypeStruct((NPAD, D), jnp.float32),
        scratch_types=[
            pltpu.VMEM((RPW, D), jnp.float32),
            pltpu.VMEM((NPAD,), jnp.float32),
            pltpu.VMEM((K25, D), jnp.float32),
            pltpu.VMEM((CHW + L,), jnp.int32),
            pltpu.VMEM((L,), jnp.int32),
            pltpu.SemaphoreType.DMA,
        ],
        compiler_params=_sc_params,
    )
    return fn(xp, w5, lists)


def _outer_body(x_ref, w5c_ref, o_ref):
    o_ref[...] = x_ref[...] * w5c_ref[...]


def _tc_outer(x2, w5c, bm=1024):
    return pl.pallas_call(
        _outer_body,
        grid=(NPAD // bm,),
        in_specs=[
            pl.BlockSpec((bm, 1), lambda i: (i, 0)),
            pl.BlockSpec((1, D), lambda i: (0, 0)),
        ],
        out_specs=pl.BlockSpec((bm, D), lambda i: (i, 0)),
        out_shape=jax.ShapeDtypeStruct((NPAD, D), jnp.float32),
    )(x2, w5c)


def _conv_body(z_ref, c_ref, wc_ref, o_ref):
    a = jnp.maximum(z_ref[...] + c_ref[...], 0.0)
    o_ref[...] = jnp.dot(a, wc_ref[...], preferred_element_type=jnp.float32)


def _tc_conv(zc, c, wc, bm=1024):
    return pl.pallas_call(
        _conv_body,
        grid=(NPAD // bm,),
        in_specs=[
            pl.BlockSpec((bm, D), lambda i: (i, 0)),
            pl.BlockSpec((bm, D), lambda i: (i, 0)),
            pl.BlockSpec((D, D), lambda i: (0, 0)),
        ],
        out_specs=pl.BlockSpec((bm, D), lambda i: (i, 0)),
        out_shape=jax.ShapeDtypeStruct((NPAD, D), jnp.float32),
    )(zc, c, wc)


def _head_body(z1, c1, z2, c2, z3, c3, z4, c4,
               wf1, bf1, wf2, bf2, o_ref, acc):
    i = pl.program_id(0)

    @pl.when(i == 0)
    def _init():
        acc[...] = jnp.zeros_like(acc)

    ys = (z1[...] + c1[...],
          z2[...] + c2[...], z3[...] + c3[...], z4[...] + c4[...])
    for idx, y in enumerate(ys):
        acc[0:1, idx * D:(idx + 1) * D] += jnp.sum(jnp.maximum(y, 0.0), axis=0, keepdims=True)

    @pl.when(i == pl.num_programs(0) - 1)
    def _final():
        p = acc[...] * (1.0 / N)
        h = jnp.maximum(
            jnp.dot(p, wf1[...], preferred_element_type=jnp.float32) + bf1[...], 0.0
        )
        o_ref[...] = jnp.dot(h, wf2[...], preferred_element_type=jnp.float32) + bf2[...]


def _tc_head(zc1, zc2, zc3, zc4, wf1, bf1, wf2, bf2, bm=1024):
    sspec = pl.BlockSpec((bm, D), lambda i: (i, 0))
    full = lambda shape: pl.BlockSpec(shape, lambda i: (0, 0))
    return pl.pallas_call(
        _head_body,
        grid=(NPAD // bm,),
        in_specs=[sspec, sspec, sspec, sspec, sspec, sspec, sspec, sspec,
                  full((512, 512)), full((1, 512)), full((512, D)), full((1, D))],
        out_specs=full((1, D)),
        out_shape=jax.ShapeDtypeStruct((1, D), jnp.float32),
        scratch_shapes=[pltpu.VMEM((1, 512), jnp.float32)],
    )(zc1[0], zc1[1], zc2[0], zc2[1], zc3[0], zc3[1], zc4[0], zc4[1],
      wf1, bf1, wf2, bf2)


def kernel(x, neigh5, neigh3d1, neigh3d2, neigh3d3, W5, W2a, W2b, W2c,
           W3a, W3b, W3c, W4a, W4b, W4c, Wf1, bf1, Wf2, bf2):
    xp = jnp.zeros((NPAD,), jnp.float32).at[:N].set(x[:, 0])
    x2 = xp.reshape(NPAD, 1)
    w5mat = W5[:, 0, :]
    w5c = w5mat[KC5:KC5 + 1]

    nsrcp = jnp.concatenate(
        [_nsrc9(n) for n in (neigh3d1, neigh3d2, neigh3d3)] + [_nsrc25(neigh5)])
    lists = _sc_compact(nsrcp)

    C1 = _sc_corr1(xp, w5mat, lists)
    Zc1 = _tc_outer(x2, w5c)

    Ws = [W2a, W2b, W2c, W3a, W3b, W3c, W4a, W4b, W4c]
    pooled = [(Zc1, C1)]
    zc = (Zc1, C1)
    for li in range(9):
        wl = Ws[li]
        wc = wl[KC]
        w8 = jnp.concatenate([wl[k] for k in range(K9) if k != KC], axis=0)
        Z = _tc_conv(zc[0], zc[1], wc)
        C = _sc_corr(zc[0], zc[1], w8, lists, li % 3)
        zc = (Z, C)
        if li in (2, 5, 8):
            pooled.append(zc)

    return _tc_head(pooled[0], pooled[1], pooled[2], pooled[3],
                    Wf1, bf1.reshape(1, 512), Wf2, bf2.reshape(1, D))
